# Initial kernel scaffold; baseline (speedup 1.0000x reference)
import jax
import jax.numpy as jnp
from jax.experimental import pallas as pl


def kernel(x, gate_w, wi_gate, wi_up, wo):
    raise NotImplementedError("write your pallas kernel here")


# speedup vs baseline: 1.3866x; 1.3866x over previous
import functools

import jax
import jax.numpy as jnp
from jax import lax
from jax.experimental import pallas as pl
from jax.experimental.pallas import tpu as pltpu
from jax.experimental.pallas import tpu_sc as plsc

N = 2048
D = 1024
FF = 1024
E = 64
CAP = 40
ROWS = (E + 1) * CAP
ZROW = E * CAP
NW = 32
TPB = N // NW


def _router_body(x_ref, gw_ref, idx_ref):
    x = x_ref[...]
    gw = gw_ref[...]
    logits = lax.dot_general(
        x, gw, (((1,), (1,)), ((), ())),
        preferred_element_type=jnp.float32)
    m = jnp.max(logits, axis=1, keepdims=True)
    cols = lax.broadcasted_iota(jnp.int32, (N, E), 1)
    eid = jnp.min(jnp.where(logits >= m, cols, E), axis=1, keepdims=True)
    oh = (cols == eid).astype(jnp.int32)
    csum = oh
    k = 1
    while k < N:
        shifted = jnp.concatenate(
            [jnp.zeros((k, E), jnp.int32), csum[:N - k]], axis=0)
        csum = csum + shifted
        k *= 2
    pos = jnp.sum(csum * oh, axis=1, keepdims=True) - 1
    g = eid * CAP + pos
    idx_ref[...] = jnp.where(pos < CAP, g, ZROW)


def _router(xf, gate_w):
    idx2 = pl.pallas_call(
        _router_body,
        out_shape=jax.ShapeDtypeStruct((N, 1), jnp.int32),
    )(xf, gate_w)
    return idx2.reshape(N)


@functools.cache
def _sc_kernels():
    mesh = plsc.VectorSubcoreMesh(core_axis_name="c", subcore_axis_name="s")
    nc = mesh.num_cores

    @functools.partial(
        pl.kernel, mesh=mesh,
        out_type=jax.ShapeDtypeStruct((ROWS, D), jnp.float32),
        scratch_types=[
            pltpu.VMEM((TPB,), jnp.int32),
            pltpu.VMEM((TPB, D), jnp.float32),
            pltpu.SemaphoreType.DMA,
        ],
    )
    def dispatch(xf_hbm, idx_hbm, out_hbm, idx_v, rows_v, sem):
        wid = lax.axis_index("s") * nc + lax.axis_index("c")
        base = wid * TPB
        pltpu.sync_copy(idx_hbm.at[pl.ds(base, TPB)], idx_v)
        pltpu.sync_copy(xf_hbm.at[pl.ds(base, TPB)], rows_v)
        pltpu.async_copy(rows_v, out_hbm.at[idx_v], sem).wait()

    @functools.partial(
        pl.kernel, mesh=mesh,
        out_type=jax.ShapeDtypeStruct((N, D), jnp.float32),
        scratch_types=[
            pltpu.VMEM((TPB,), jnp.int32),
            pltpu.VMEM((TPB, D), jnp.float32),
            pltpu.SemaphoreType.DMA,
        ],
    )
    def combine(eo_hbm, idx_hbm, y_hbm, idx_v, rows_v, sem):
        wid = lax.axis_index("s") * nc + lax.axis_index("c")
        base = wid * TPB
        pltpu.sync_copy(idx_hbm.at[pl.ds(base, TPB)], idx_v)
        pltpu.async_copy(eo_hbm.at[idx_v], rows_v, sem).wait()
        pltpu.sync_copy(rows_v, y_hbm.at[pl.ds(base, TPB)])

    return dispatch, combine


def _ffn_body(disp_ref, wg_ref, wu_ref, wo_ref, out_ref):
    e = pl.program_id(0)

    @pl.when(e < E)
    def _compute():
        xb = disp_ref[...]
        wg = wg_ref[0]
        wu = wu_ref[0]
        wo = wo_ref[0]
        hg = lax.dot_general(xb, wg, (((1,), (1,)), ((), ())),
                             preferred_element_type=jnp.float32)
        hu = lax.dot_general(xb, wu, (((1,), (1,)), ((), ())),
                             preferred_element_type=jnp.float32)
        h = hg * jax.nn.sigmoid(hg) * hu
        out_ref[...] = lax.dot_general(h, wo, (((1,), (1,)), ((), ())),
                                       preferred_element_type=jnp.float32)

    @pl.when(e == E)
    def _zero():
        out_ref[...] = jnp.zeros((CAP, D), jnp.float32)


def _ffn(dispatched, wi_gate, wi_up, wo):
    return pl.pallas_call(
        _ffn_body,
        grid=(E + 1,),
        in_specs=[
            pl.BlockSpec((CAP, D), lambda e: (e, 0)),
            pl.BlockSpec((1, FF, D), lambda e: (jnp.minimum(e, E - 1), 0, 0)),
            pl.BlockSpec((1, FF, D), lambda e: (jnp.minimum(e, E - 1), 0, 0)),
            pl.BlockSpec((1, D, FF), lambda e: (jnp.minimum(e, E - 1), 0, 0)),
        ],
        out_specs=pl.BlockSpec((CAP, D), lambda e: (e, 0)),
        out_shape=jax.ShapeDtypeStruct((ROWS, D), jnp.float32),
    )(dispatched, wi_gate, wi_up, wo)


def kernel(x, gate_w, wi_gate, wi_up, wo):
    B, S, D_ = x.shape
    xf = x.reshape(N, D)
    dispatch, combine = _sc_kernels()
    idx = _router(xf, gate_w)
    dispatched = dispatch(xf, idx)
    eo = _ffn(dispatched, wi_gate, wi_up, wo)
    y = combine(eo, idx)
    return y.reshape(B, S, D_)
